# baseline (speedup 1.0000x reference)
import functools

import jax
import jax.numpy as jnp
from jax import lax
from jax.experimental import pallas as pl
from jax.experimental.pallas import tpu as pltpu
from jax.experimental.pallas import tpu_sc as plsc

V, D = 1000000, 32
N, K = 16384, 20
B = N * K
NC, NS = 2, 16
NW = NC * NS
LANES = 16

_mesh = plsc.VectorSubcoreMesh(core_axis_name="c", subcore_axis_name="s")

JBLK = 128
NJ = (V + JBLK - 1) // JBLK
QPB = JBLK * D // 128
NFULL = NJ // NW
NREM = NJ - NFULL * NW
VLAST = (V - (NJ - 1) * JBLK) * D // 128
SKEW = 33


@functools.partial(
    pl.kernel,
    mesh=_mesh,
    out_type=jax.ShapeDtypeStruct((V * D // 128, 128), jnp.float32),
    scratch_types=[
        pltpu.VMEM((2, D, JBLK), jnp.float32),
        pltpu.VMEM((2, QPB, 128), jnp.float32),
        pltpu.VMEM((JBLK * SKEW,), jnp.float32),
        pltpu.SemaphoreType.DMA,
        pltpu.SemaphoreType.DMA,
        pltpu.SemaphoreType.DMA,
        pltpu.SemaphoreType.DMA,
    ],
    compiler_params=pltpu.CompilerParams(
        use_tc_tiling_on_sc=True, needs_layout_passes=False),
)
def _pack_kernel(tabt_hbm, rm_hbm, tbuf, pbuf, skew,
                 sem_i0, sem_i1, sem_o0, sem_o1):
    wid = lax.axis_index("s") * NC + lax.axis_index("c")
    sem_i = (sem_i0, sem_i1)
    sem_o = (sem_o0, sem_o1)
    iota = lax.iota(jnp.int32, LANES)
    skew_a = iota * SKEW

    def transpose_block(p):
        for d in range(D):
            for w in range(JBLK // LANES):
                vec = tbuf[p, d, pl.ds(w * LANES, LANES)]
                plsc.store_scatter(skew, [skew_a + (w * LANES * SKEW + d)], vec)
        for q in range(QPB):
            for h in range(128 // LANES):
                l = 4 * q + h // 2
                base = l * SKEW + (h % 2) * LANES
                vec = plsc.load_gather(skew, [iota + base])
                pbuf[p, q, pl.ds(h * LANES, LANES)] = vec

    def start_in(i, p):
        j = (wid + i * NW) * JBLK
        return pltpu.async_copy(
            tabt_hbm.at[:, pl.ds(j, JBLK)], tbuf.at[p], sem_i[p])

    def wait_in(p):
        pltpu.make_async_copy(
            tabt_hbm.at[:, pl.ds(0, JBLK)], tbuf.at[p], sem_i[p]).wait()

    def start_out(i, p):
        q = (wid + i * NW) * QPB
        return pltpu.async_copy(
            pbuf.at[p], rm_hbm.at[pl.ds(q, QPB)], sem_o[p])

    def wait_out(p):
        pltpu.make_async_copy(
            pbuf.at[p], rm_hbm.at[pl.ds(0, QPB)], sem_o[p]).wait()

    start_in(0, 0)
    start_in(1, 1)

    def body(i2, _):
        i = i2 * 2
        for p in (0, 1):
            wait_in(p)

            @pl.when(i + p >= 2)
            def _():
                wait_out(p)

            transpose_block(p)
            start_out(i + p, p)

            @pl.when(i + p + 2 < NFULL)
            def _():
                start_in(i + p + 2, p)

        return _

    lax.fori_loop(0, NFULL // 2, body, None)
    wait_out(0)
    wait_out(1)

    @pl.when(wid < NREM)
    def _():
        j = (NFULL * NW + wid) * JBLK
        pltpu.sync_copy(tabt_hbm.at[:, pl.ds(j, JBLK)], tbuf.at[0])
        transpose_block(0)
        q = (NFULL * NW + wid) * QPB

        @pl.when(wid < NREM - 1)
        def _():
            pltpu.sync_copy(pbuf.at[0], rm_hbm.at[pl.ds(q, QPB)])

        @pl.when(wid == NREM - 1)
        def _():
            pltpu.sync_copy(pbuf.at[0, pl.ds(0, VLAST)],
                            rm_hbm.at[pl.ds(q, VLAST)])


B_PER_W = B // NW
CH = 1024
NCHUNK = B_PER_W // CH


@functools.partial(
    pl.kernel,
    mesh=_mesh,
    out_type=jax.ShapeDtypeStruct((B, D), jnp.float32),
    scratch_types=[
        pltpu.VMEM((NCHUNK, CH), jnp.int32),
        pltpu.VMEM((2, CH, D), jnp.float32),
        pltpu.SemaphoreType.DMA,
        pltpu.SemaphoreType.DMA,
        pltpu.SemaphoreType.DMA,
        pltpu.SemaphoreType.DMA,
    ],
    compiler_params=pltpu.CompilerParams(use_tc_tiling_on_sc=False),
)
def _gather_kernel(idx_hbm, table_hbm, out_hbm, idx_v, rows_v,
                   sem_g0, sem_g1, sem_w0, sem_w1):
    wid = lax.axis_index("s") * NC + lax.axis_index("c")
    base = wid * B_PER_W
    sem_g = (sem_g0, sem_g1)
    sem_w = (sem_w0, sem_w1)

    pltpu.sync_copy(idx_hbm.at[wid], idx_v)

    h_g = [None, None]
    h_w = [None, None]
    h_g[0] = pltpu.async_copy(table_hbm.at[idx_v.at[0]], rows_v.at[0], sem_g[0])
    for c in range(NCHUNK):
        b = c % 2
        nb = (c + 1) % 2
        if c + 1 < NCHUNK:
            if h_w[nb] is not None:
                h_w[nb].wait()
                h_w[nb] = None
            h_g[nb] = pltpu.async_copy(
                table_hbm.at[idx_v.at[c + 1]], rows_v.at[nb], sem_g[nb])
        h_g[b].wait()
        h_w[b] = pltpu.async_copy(
            rows_v.at[b], out_hbm.at[pl.ds(base + c * CH, CH)], sem_w[b])
    for b in range(2):
        if h_w[b] is not None:
            h_w[b].wait()


def kernel(index, table):
    rm = _pack_kernel(table.T)
    tab_lin = rm.reshape(V, D)
    idx = index.reshape(-1).astype(jnp.int32).reshape(NW, NCHUNK, CH)
    out = _gather_kernel(idx, tab_lin)
    return out.reshape(index.shape[0], index.shape[1], D)


# speedup vs baseline: 1.2937x; 1.2937x over previous
import functools

import jax
import jax.numpy as jnp
from jax import lax
from jax.experimental import pallas as pl
from jax.experimental.pallas import tpu as pltpu
from jax.experimental.pallas import tpu_sc as plsc

B = 16384 * 20
D = 32
NC, NS = 2, 16
NW = NC * NS
B_PER_W = B // NW
CH = 1024
NCHUNK = B_PER_W // CH

_mesh = plsc.VectorSubcoreMesh(core_axis_name="c", subcore_axis_name="s")


@functools.partial(
    pl.kernel,
    mesh=_mesh,
    out_type=jax.ShapeDtypeStruct((B, D), jnp.float32),
    scratch_types=[
        pltpu.VMEM((NCHUNK, CH), jnp.int32),
        pltpu.VMEM((2, CH, D), jnp.float32),
        pltpu.SemaphoreType.DMA,
        pltpu.SemaphoreType.DMA,
        pltpu.SemaphoreType.DMA,
        pltpu.SemaphoreType.DMA,
    ],
    compiler_params=pltpu.CompilerParams(use_tc_tiling_on_sc=False),
)
def _gather_kernel(idx_hbm, table_hbm, out_hbm, idx_v, rows_v,
                   sem_g0, sem_g1, sem_w0, sem_w1):
    wid = lax.axis_index("s") * NC + lax.axis_index("c")
    base = wid * B_PER_W
    sem_g = (sem_g0, sem_g1)
    sem_w = (sem_w0, sem_w1)

    pltpu.sync_copy(idx_hbm.at[wid], idx_v)

    h_g = [None, None]
    h_w = [None, None]
    h_g[0] = pltpu.async_copy(table_hbm.at[idx_v.at[0]], rows_v.at[0], sem_g[0])
    for c in range(NCHUNK):
        b = c % 2
        nb = (c + 1) % 2
        if c + 1 < NCHUNK:
            if h_w[nb] is not None:
                h_w[nb].wait()
                h_w[nb] = None
            h_g[nb] = pltpu.async_copy(
                table_hbm.at[idx_v.at[c + 1]], rows_v.at[nb], sem_g[nb])
        h_g[b].wait()
        h_w[b] = pltpu.async_copy(
            rows_v.at[b], out_hbm.at[pl.ds(base + c * CH, CH)], sem_w[b])
    for b in range(2):
        if h_w[b] is not None:
            h_w[b].wait()


def kernel(index, table):
    idx = index.reshape(-1).astype(jnp.int32).reshape(NW, NCHUNK, CH)
    out = _gather_kernel(idx, table)
    return out.reshape(index.shape[0], index.shape[1], D)
